# baseline (speedup 1.0000x reference)
import functools

import jax
import jax.numpy as jnp
from jax import lax
from jax.experimental import pallas as pl
from jax.experimental.pallas import tpu as pltpu
from jax.experimental.pallas import tpu_sc as plsc


def _sc_geometry():
    try:
        info = plsc.get_sparse_core_info()
        return info.num_cores, info.num_subcores
    except Exception:
        return 2, 16


def _embed_lookup(x2d, table, pe):
    B, S = x2d.shape
    V, D = table.shape
    NC, NS = _sc_geometry()
    NW = NC * NS
    C = S // NW

    mesh = plsc.VectorSubcoreMesh(core_axis_name="c", subcore_axis_name="s")

    NBUF = 4
    A = 3
    H = C // NBUF

    @functools.partial(
        pl.kernel,
        mesh=mesh,
        out_type=jax.ShapeDtypeStruct((B, S, D), jnp.float32),
        scratch_types=[
            pltpu.VMEM((B, C), jnp.int32),
            [pltpu.VMEM((H, D), jnp.float32)] * NBUF,
            pltpu.VMEM((C, D), jnp.float32),
            [pltpu.SemaphoreType.DMA] * NBUF,
            [pltpu.SemaphoreType.DMA] * NBUF,
            pltpu.SemaphoreType.DMA,
        ],
    )
    def emb(x_hbm, table_hbm, pe_hbm, out_hbm, idx_v, rows_v, pe_v, gsem, wsem,
            ssem):
        wid = lax.axis_index("s") * NC + lax.axis_index("c")
        base = wid * C
        nj = D // 16
        descs = [pltpu.async_copy(x_hbm.at[b, pl.ds(base, C)], idx_v.at[b], ssem)
                 for b in range(B)]
        descs.append(pltpu.async_copy(pe_hbm.at[pl.ds(base, C)], pe_v, ssem))

        def start_gather(b, h, p):
            return pltpu.async_copy(
                table_hbm.at[idx_v.at[b, pl.ds(h * H, H)]], rows_v[p], gsem[p])

        def drain_gather(p):
            pltpu.make_async_copy(
                table_hbm.at[idx_v.at[0, pl.ds(0, H)]], rows_v[p],
                gsem[p]).wait()

        def drain_write(p):
            pltpu.make_async_copy(
                rows_v[p], out_hbm.at[0, pl.ds(base, H)], wsem[p]).wait()

        for d in descs:
            d.wait()
        for k in range(A):
            start_gather(0, k, k)

        def group(g, carry):
            for k in range(NBUF):
                drain_gather(k)
                if k + A < NBUF:
                    @pl.when(g > 0)
                    def _():
                        drain_write(k + A)
                    start_gather(g, k + A, k + A)
                else:
                    @pl.when(g < B - 1)
                    def _():
                        drain_write(k + A - NBUF)
                        start_gather(g + 1, k + A - NBUF, k + A - NBUF)

                def add_pe_row(r, _k=k):
                    for j in range(nj):
                        plsc.addupdate(rows_v[_k].at[r, pl.ds(j * 16, 16)],
                                       pe_v[_k * H + r, pl.ds(j * 16, 16)])

                plsc.parallel_loop(0, H, 1, unroll=2)(add_pe_row)
                pltpu.async_copy(
                    rows_v[k], out_hbm.at[g, pl.ds(base + k * H, H)], wsem[k])
            return carry

        lax.fori_loop(0, B, group, 0)
        for k in range(NBUF):
            drain_write(k)

    return emb(x2d, table, pe)


def kernel(x, table, pe):
    return _embed_lookup(x.astype(jnp.int32), table, pe.astype(jnp.float32))


# speedup vs baseline: 1.1445x; 1.1445x over previous
import functools

import jax
import jax.numpy as jnp
from jax import lax
from jax.experimental import pallas as pl
from jax.experimental.pallas import tpu as pltpu
from jax.experimental.pallas import tpu_sc as plsc


def _sc_geometry():
    try:
        info = plsc.get_sparse_core_info()
        return info.num_cores, info.num_subcores
    except Exception:
        return 2, 16


def _embed_lookup(x2d, table, pe):
    B, S = x2d.shape
    V, D = table.shape
    NC, NS = _sc_geometry()
    NW = NC * NS
    C = S // NW

    mesh = plsc.VectorSubcoreMesh(core_axis_name="c", subcore_axis_name="s")

    NCH = 8
    H = C // NCH
    NP = 3
    A = 2

    @functools.partial(
        pl.kernel,
        mesh=mesh,
        out_type=jax.ShapeDtypeStruct((B, S, D), jnp.float32),
        scratch_types=[
            pltpu.VMEM((B, C), jnp.int32),
            [[pltpu.VMEM((H, D), jnp.float32) for _ in range(B)]
             for _ in range(NP)],
            pltpu.VMEM((C, D), jnp.float32),
            [pltpu.SemaphoreType.DMA] * NP,
            [pltpu.SemaphoreType.DMA] * NP,
            pltpu.SemaphoreType.DMA,
        ],
    )
    def emb(x_hbm, table_hbm, pe_hbm, out_hbm, idx_v, rows_v, pe_v, gsem, wsem,
            ssem):
        wid = lax.axis_index("s") * NC + lax.axis_index("c")
        base = wid * C
        nj = D // 16
        descs = [pltpu.async_copy(x_hbm.at[b, pl.ds(base, C)], idx_v.at[b], ssem)
                 for b in range(B)]
        descs.append(pltpu.async_copy(pe_hbm.at[pl.ds(base, C)], pe_v, ssem))

        def start_gathers(g):
            p = g % NP
            for b in range(B):
                pltpu.async_copy(
                    table_hbm.at[idx_v.at[b, pl.ds(g * H, H)]],
                    rows_v[p][b], gsem[p])

        def drain_gathers(g):
            p = g % NP
            for b in range(B):
                pltpu.make_async_copy(
                    table_hbm.at[idx_v.at[0, pl.ds(0, H)]], rows_v[p][b],
                    gsem[p]).wait()

        def start_writes(g):
            p = g % NP
            for b in range(B):
                pltpu.async_copy(
                    rows_v[p][b], out_hbm.at[b, pl.ds(base + g * H, H)],
                    wsem[p])

        def drain_writes(g):
            p = g % NP
            for b in range(B):
                pltpu.make_async_copy(
                    rows_v[p][b], out_hbm.at[0, pl.ds(base, H)],
                    wsem[p]).wait()

        for d in descs:
            d.wait()
        start_gathers(0)
        start_gathers(1)
        for g in range(NCH):
            drain_gathers(g)
            if g + A < NCH:
                if g + A - NP >= 0:
                    drain_writes(g + A - NP)
                start_gathers(g + A)

            p = g % NP

            def add_pe_row(r, _g=g, _p=p):
                for j in range(nj):
                    v = pe_v[_g * H + r, pl.ds(j * 16, 16)]
                    for b in range(B):
                        plsc.addupdate(
                            rows_v[_p][b].at[r, pl.ds(j * 16, 16)], v)

            plsc.parallel_loop(0, H, 1, unroll=1)(add_pe_row)
            start_writes(g)
        for g in range(NCH - NP, NCH):
            drain_writes(g)

    return emb(x2d, table, pe)


def kernel(x, table, pe):
    return _embed_lookup(x.astype(jnp.int32), table, pe.astype(jnp.float32))
